# Initial kernel scaffold; baseline (speedup 1.0000x reference)
import jax
import jax.numpy as jnp
from jax.experimental import pallas as pl


def kernel(token_idx, multi_idx, one_idx, word_emb, multi_tables, one_tables, wide_multi, wide_one, mlp_Ws, mlp_bs, W_last, b_last):
    raise NotImplementedError("write your pallas kernel here")


# speedup vs baseline: 1.9939x; 1.9939x over previous
import functools

import jax
import jax.numpy as jnp
from jax import lax
from jax.experimental import pallas as pl
from jax.experimental.pallas import tpu as pltpu
from jax.experimental.pallas import tpu_sc as plsc

BATCH = 4096
L = 20
N_MULTI = 6
N_ONE = 20
VOCAB = 100000
WORD_DIM = 128
EMB = 32

NC, NS = 2, 16
NW = NC * NS
SPW = BATCH // NW
SPW20 = SPW * L

TOK_CH = 8
TOK_CS = SPW // TOK_CH
MUL_CH = 2
MUL_CS = SPW // MUL_CH


def _sc_body(tok_ref, mid_ref, oid_ref, wemb_ref, mt_ref, ot_ref, wm_ref,
             wo_ref, emb_out, wmv_out, wov_out,
             idxw, idx1, tbuf, tstage, mbuf, mstage, obuf, wvm, wvo,
             sem, semw):
    wid = lax.axis_index("subcore") * NC + lax.axis_index("core")
    base = wid * SPW
    base20 = base * L

    pltpu.sync_copy(tok_ref.at[pl.ds(base20, SPW20)], idxw)

    def tok_chunk(c, _):
        pltpu.async_copy(
            wemb_ref.at[idxw.at[pl.ds(c * (TOK_CS * L), TOK_CS * L)]],
            tbuf, sem).wait()

        def tok_sample(s, _):
            row = s * L
            for v in range(WORD_DIM // 16):
                acc = tbuf[row, pl.ds(v * 16, 16)]
                for l in range(1, L):
                    acc = acc + tbuf[row + l, pl.ds(v * 16, 16)]
                tstage[c * TOK_CS + s, pl.ds(v * 16, 16)] = acc * (1.0 / L)
            return 0

        lax.fori_loop(0, TOK_CS, tok_sample, 0)
        return 0

    lax.fori_loop(0, TOK_CH, tok_chunk, 0)
    pltpu.sync_copy(tstage, emb_out.at[pl.ds(base, SPW), pl.ds(0, WORD_DIM)])

    def multi_table(i, _):
        pltpu.sync_copy(mid_ref.at[pl.ds(i * (BATCH * L) + base20, SPW20)],
                        idxw)

        def add_off(k, _):
            idxw[pl.ds(k * 16, 16)] = idxw[pl.ds(k * 16, 16)] + i * VOCAB
            return 0

        lax.fori_loop(0, SPW20 // 16, add_off, 0)

        wcopy = pltpu.async_copy(wm_ref.at[idxw], wvm, semw)

        def m_chunk(c, _):
            pltpu.async_copy(
                mt_ref.at[idxw.at[pl.ds(c * (MUL_CS * L), MUL_CS * L)]],
                mbuf, sem).wait()

            def m_sample(s, _):
                row = s * L
                for v in range(EMB // 16):
                    acc = mbuf[row, pl.ds(v * 16, 16)]
                    for l in range(1, L):
                        acc = acc + mbuf[row + l, pl.ds(v * 16, 16)]
                    mstage[c * MUL_CS + s, pl.ds(v * 16, 16)] = acc * (1.0 / L)
                return 0

            lax.fori_loop(0, MUL_CS, m_sample, 0)
            return 0

        lax.fori_loop(0, MUL_CH, m_chunk, 0)
        pltpu.sync_copy(mstage,
                        emb_out.at[pl.ds(base, SPW),
                                   pl.ds(WORD_DIM + EMB * i, EMB)])
        wcopy.wait()
        pltpu.sync_copy(wvm, wmv_out.at[i, pl.ds(base20, SPW20)])
        return 0

    lax.fori_loop(0, N_MULTI, multi_table, 0)

    def one_table(j, _):
        pltpu.sync_copy(oid_ref.at[pl.ds(j * BATCH + base, SPW)], idx1)

        def add_off(k, _):
            idx1[pl.ds(k * 16, 16)] = idx1[pl.ds(k * 16, 16)] + j * VOCAB
            return 0

        lax.fori_loop(0, SPW // 16, add_off, 0)
        pltpu.async_copy(ot_ref.at[idx1], obuf, sem).wait()
        pltpu.sync_copy(obuf,
                        emb_out.at[pl.ds(base, SPW),
                                   pl.ds(WORD_DIM + EMB * N_MULTI + EMB * j,
                                         EMB)])
        pltpu.async_copy(wo_ref.at[idx1], wvo, semw).wait()
        pltpu.sync_copy(wvo, wov_out.at[j, pl.ds(base, SPW)])
        return 0

    lax.fori_loop(0, N_ONE, one_table, 0)


def _sc_embed(tok, mid, oid, wemb, mt, ot, wm, wo):
    mesh = plsc.VectorSubcoreMesh(core_axis_name="core",
                                  subcore_axis_name="subcore",
                                  num_cores=NC, num_subcores=NS)
    out_type = (
        jax.ShapeDtypeStruct((BATCH, WORD_DIM + EMB * (N_MULTI + N_ONE)),
                             jnp.float32),
        jax.ShapeDtypeStruct((N_MULTI, BATCH * L), jnp.float32),
        jax.ShapeDtypeStruct((N_ONE, BATCH), jnp.float32),
    )
    scratch = [
        pltpu.VMEM((SPW20,), jnp.int32),
        pltpu.VMEM((SPW,), jnp.int32),
        pltpu.VMEM((TOK_CS * L, WORD_DIM), jnp.float32),
        pltpu.VMEM((SPW, WORD_DIM), jnp.float32),
        pltpu.VMEM((MUL_CS * L, EMB), jnp.float32),
        pltpu.VMEM((SPW, EMB), jnp.float32),
        pltpu.VMEM((SPW, EMB), jnp.float32),
        pltpu.VMEM((SPW20,), jnp.float32),
        pltpu.VMEM((SPW,), jnp.float32),
        pltpu.SemaphoreType.DMA,
        pltpu.SemaphoreType.DMA,
    ]
    fn = pl.kernel(_sc_body, mesh=mesh, out_type=out_type,
                   scratch_types=scratch,
                   compiler_params=pltpu.CompilerParams(
                       use_tc_tiling_on_sc=False))
    return fn(tok, mid, oid, wemb, mt, ot, wm, wo)


def _mlp_body(x_ref, w0, b0, w1, b1, w2, b2, wl, bl, wm, wo, o_ref):
    h = jnp.maximum(
        jnp.dot(x_ref[...], w0[...], preferred_element_type=jnp.float32)
        + b0[...], 0.0)
    h = jnp.maximum(
        jnp.dot(h, w1[...], preferred_element_type=jnp.float32) + b1[...],
        0.0)
    h = jnp.maximum(
        jnp.dot(h, w2[...], preferred_element_type=jnp.float32) + b2[...],
        0.0)
    wide = jnp.sum(wm[...], axis=(0, 2)) + jnp.sum(wo[...], axis=0)
    z = (jnp.dot(h, wl[...], preferred_element_type=jnp.float32) + bl[...]
         + wide[:, None])
    o_ref[...] = jax.nn.sigmoid(z)


def _mlp(emb, W0, b0, W1, b1, W2, b2, Wl, bl, wmv, wov):
    BB = 512
    grid = (BATCH // BB,)
    D = emb.shape[1]
    return pl.pallas_call(
        _mlp_body,
        grid=grid,
        in_specs=[
            pl.BlockSpec((BB, D), lambda i: (i, 0)),
            pl.BlockSpec(W0.shape, lambda i: (0, 0)),
            pl.BlockSpec(b0.shape, lambda i: (0, 0)),
            pl.BlockSpec(W1.shape, lambda i: (0, 0)),
            pl.BlockSpec(b1.shape, lambda i: (0, 0)),
            pl.BlockSpec(W2.shape, lambda i: (0, 0)),
            pl.BlockSpec(b2.shape, lambda i: (0, 0)),
            pl.BlockSpec(Wl.shape, lambda i: (0, 0)),
            pl.BlockSpec(bl.shape, lambda i: (0, 0)),
            pl.BlockSpec((N_MULTI, BB, L), lambda i: (0, i, 0)),
            pl.BlockSpec((N_ONE, BB), lambda i: (0, i)),
        ],
        out_specs=pl.BlockSpec((BB, 1), lambda i: (i, 0)),
        out_shape=jax.ShapeDtypeStruct((BATCH, 1), jnp.float32),
    )(emb, W0, b0, W1, b1, W2, b2, Wl, bl, wmv, wov)


def kernel(token_idx, multi_idx, one_idx, word_emb, multi_tables, one_tables,
           wide_multi, wide_one, mlp_Ws, mlp_bs, W_last, b_last):
    tok = token_idx.astype(jnp.int32).reshape(-1)
    mid = multi_idx.astype(jnp.int32).reshape(-1)
    oid = one_idx.astype(jnp.int32).reshape(-1)
    mt = multi_tables.reshape(N_MULTI * VOCAB, EMB)
    ot = one_tables.reshape(N_ONE * VOCAB, EMB)
    wm = wide_multi.reshape(N_MULTI * VOCAB)
    wo = wide_one.reshape(N_ONE * VOCAB)

    emb, wmv, wov = _sc_embed(tok, mid, oid, word_emb, mt, ot, wm, wo)

    W0, W1, W2 = mlp_Ws
    b0, b1, b2 = (b.reshape(1, -1) for b in mlp_bs)
    return _mlp(emb, W0, b0, W1, b1, W2, b2, W_last, b_last.reshape(1, 1),
                wmv.reshape(N_MULTI, BATCH, L), wov.reshape(N_ONE, BATCH))
